# baseline (speedup 1.0000x reference)
import functools

import jax
import jax.numpy as jnp
from jax import lax
from jax.experimental import pallas as pl
from jax.experimental.pallas import tpu as pltpu
from jax.experimental.pallas import tpu_sc as plsc

_B_TOTAL = 4096 * 200
_D = 32
_NW = 32
_ROWS_PER_W = 200
_K = 4
_CHUNK = _K * 128
_N_CHUNKS = _ROWS_PER_W // _K


def _transpose_chunk(gbuf, tbuf, iota16):
    def k_body(k, _):
        rowb = iota16 + k * 128
        rows_j = [rowb + (j * 16) for j in range(8)]

        def td_body(td, __):
            obase = k * 4096 + td * 1024
            colb = td * 8
            for d8 in range(8):
                cols = jnp.zeros((16,), jnp.int32) + (colb + d8)
                for j in range(8):
                    v = plsc.load_gather(gbuf, [rows_j[j], cols])
                    tbuf[pl.ds(obase + d8 * 128 + j * 16, 16)] = v
            return __

        return lax.fori_loop(0, 4, td_body, _)

    lax.fori_loop(0, 4, k_body, 0)


def _emb_body(ids_hbm, table_hbm, out_hbm, ids_v, g0, g1, t0, t1, semg, semw):
    wid = lax.axis_index("s") * 2 + lax.axis_index("c")
    base_row = wid * _ROWS_PER_W
    iota16 = jnp.arange(16, dtype=jnp.int32)

    pltpu.sync_copy(ids_hbm.at[pl.ds(base_row * 128, _ROWS_PER_W * 128)], ids_v)

    def gather(c, buf):
        pltpu.async_copy(
            table_hbm.at[ids_v.at[pl.ds(c * _CHUNK, _CHUNK)]], buf, semg)

    def writeback(c, tbuf):
        for k in range(_K):
            r = base_row + c * _K + k
            st = r // 256
            bt = (r // 8) % 32
            s = st * 8 + r % 8
            for td in range(4):
                pltpu.async_copy(
                    tbuf.at[pl.ds(k * 4096 + td * 1024, 1024)],
                    out_hbm.at[s, td, bt], semw)

    def drain_one_chunk():
        pltpu.make_async_copy(table_hbm.at[pl.ds(0, _CHUNK)], g0, semw).wait()

    def wait_gather(buf):
        pltpu.make_async_copy(
            table_hbm.at[ids_v.at[pl.ds(0, _CHUNK)]], buf, semg).wait()

    def process(c, gbuf, tbuf):
        wait_gather(gbuf)
        _transpose_chunk(gbuf, tbuf, iota16)
        writeback(c, tbuf)

    gather(0, g0)
    gather(1, g1)
    process(0, g0, t0)
    gather(2, g0)
    process(1, g1, t1)
    gather(3, g1)

    def steady(m, carry):
        c0 = 2 * m + 2
        drain_one_chunk()
        process(c0, g0, t0)
        gather(c0 + 2, g0)
        drain_one_chunk()
        process(c0 + 1, g1, t1)
        gather(c0 + 3, g1)
        return carry

    lax.fori_loop(0, (_N_CHUNKS - 4) // 2, steady, 0)

    drain_one_chunk()
    process(_N_CHUNKS - 2, g0, t0)
    drain_one_chunk()
    process(_N_CHUNKS - 1, g1, t1)
    drain_one_chunk()
    drain_one_chunk()


@jax.jit
def _embedding_lookup(ids_flat, weight):
    mesh = plsc.VectorSubcoreMesh(core_axis_name="c", subcore_axis_name="s")
    f = functools.partial(
        pl.kernel,
        mesh=mesh,
        out_type=jax.ShapeDtypeStruct((200, 4, 32, 1024), jnp.float32),
        scratch_types=[
            pltpu.VMEM((_ROWS_PER_W * 128,), jnp.int32),
            pltpu.VMEM((_CHUNK, _D), jnp.float32),
            pltpu.VMEM((_CHUNK, _D), jnp.float32),
            pltpu.VMEM((_K * 4096,), jnp.float32),
            pltpu.VMEM((_K * 4096,), jnp.float32),
            pltpu.SemaphoreType.DMA,
            pltpu.SemaphoreType.DMA,
        ],
        compiler_params=pltpu.CompilerParams(
            use_tc_tiling_on_sc=False, needs_layout_passes=False),
    )(_emb_body)
    return f(ids_flat, weight)


def kernel(input_ids, weight):
    ids1 = (
        input_ids.T.reshape(25, 8, 32, 128)
        .transpose(0, 2, 1, 3)
        .reshape(_B_TOTAL)
        .astype(jnp.int32)
    )
    out4 = _embedding_lookup(ids1, weight)
    out5 = out4.reshape(200, 4, 32, 8, 128)
    return jnp.transpose(out5, (2, 4, 0, 1, 3)).reshape(4096, 200, 32)


# speedup vs baseline: 1.1431x; 1.1431x over previous
import functools

import jax
import jax.numpy as jnp
from jax import lax
from jax.experimental import pallas as pl
from jax.experimental.pallas import tpu as pltpu
from jax.experimental.pallas import tpu_sc as plsc

_B_TOTAL = 4096 * 200
_D = 32
_NW = 32
_ROWS_PER_W = 200
_K = 4
_CHUNK = _K * 128
_N_CHUNKS = _ROWS_PER_W // _K


def _transpose_chunk(gbuf, tbuf, voff_lo, voff_hi):
    def r_body(r, _):
        base = (r // 128) * 4096 + (r % 128)
        lo = gbuf[r, pl.ds(0, 16)]
        hi = gbuf[r, pl.ds(16, 16)]
        plsc.store_scatter(tbuf, [voff_lo + base], lo)
        plsc.store_scatter(tbuf, [voff_hi + base], hi)
        return _

    lax.fori_loop(0, _CHUNK, r_body, 0)


def _emb_body(ids_hbm, table_hbm, out_hbm, ids_v, g0, g1, t0, t1, semg, semw):
    wid = lax.axis_index("s") * 2 + lax.axis_index("c")
    base_row = wid * _ROWS_PER_W
    iota16 = jnp.arange(16, dtype=jnp.int32)
    voff_lo = (iota16 // 8) * 1024 + (iota16 % 8) * 128
    voff_hi = voff_lo + 2048

    pltpu.sync_copy(ids_hbm.at[pl.ds(base_row * 128, _ROWS_PER_W * 128)], ids_v)

    def gather(c, buf):
        pltpu.async_copy(
            table_hbm.at[ids_v.at[pl.ds(c * _CHUNK, _CHUNK)]], buf, semg)

    def writeback(c, tbuf):
        for k in range(_K):
            r = base_row + c * _K + k
            st = r // 256
            bt = (r // 8) % 32
            s = st * 8 + r % 8
            for td in range(4):
                pltpu.async_copy(
                    tbuf.at[pl.ds(k * 4096 + td * 1024, 1024)],
                    out_hbm.at[s, td, bt], semw)

    def drain_one_chunk():
        pltpu.make_async_copy(table_hbm.at[pl.ds(0, _CHUNK)], g0, semw).wait()

    def wait_gather(buf):
        pltpu.make_async_copy(
            table_hbm.at[ids_v.at[pl.ds(0, _CHUNK)]], buf, semg).wait()

    def process(c, gbuf, tbuf):
        wait_gather(gbuf)
        _transpose_chunk(gbuf, tbuf, voff_lo, voff_hi)
        writeback(c, tbuf)

    gather(0, g0)
    gather(1, g1)
    process(0, g0, t0)
    gather(2, g0)
    process(1, g1, t1)
    gather(3, g1)

    def steady(m, carry):
        c0 = 2 * m + 2
        drain_one_chunk()
        process(c0, g0, t0)
        gather(c0 + 2, g0)
        drain_one_chunk()
        process(c0 + 1, g1, t1)
        gather(c0 + 3, g1)
        return carry

    lax.fori_loop(0, (_N_CHUNKS - 4) // 2, steady, 0)

    drain_one_chunk()
    process(_N_CHUNKS - 2, g0, t0)
    drain_one_chunk()
    process(_N_CHUNKS - 1, g1, t1)
    drain_one_chunk()
    drain_one_chunk()


@jax.jit
def _embedding_lookup(ids_flat, weight):
    mesh = plsc.VectorSubcoreMesh(core_axis_name="c", subcore_axis_name="s")
    f = functools.partial(
        pl.kernel,
        mesh=mesh,
        out_type=jax.ShapeDtypeStruct((200, 4, 32, 1024), jnp.float32),
        scratch_types=[
            pltpu.VMEM((_ROWS_PER_W * 128,), jnp.int32),
            pltpu.VMEM((_CHUNK, _D), jnp.float32),
            pltpu.VMEM((_CHUNK, _D), jnp.float32),
            pltpu.VMEM((_K * 4096,), jnp.float32),
            pltpu.VMEM((_K * 4096,), jnp.float32),
            pltpu.SemaphoreType.DMA,
            pltpu.SemaphoreType.DMA,
        ],
        compiler_params=pltpu.CompilerParams(
            use_tc_tiling_on_sc=False, needs_layout_passes=False),
    )(_emb_body)
    return f(ids_flat, weight)


def kernel(input_ids, weight):
    ids1 = (
        input_ids.T.reshape(25, 8, 32, 128)
        .transpose(0, 2, 1, 3)
        .reshape(_B_TOTAL)
        .astype(jnp.int32)
    )
    out4 = _embedding_lookup(ids1, weight)
    out5 = out4.reshape(200, 4, 32, 8, 128)
    return jnp.transpose(out5, (2, 4, 0, 1, 3)).reshape(4096, 200, 32)


# speedup vs baseline: 1.1549x; 1.0103x over previous
import functools

import jax
import jax.numpy as jnp
from jax import lax
from jax.experimental import pallas as pl
from jax.experimental.pallas import tpu as pltpu
from jax.experimental.pallas import tpu_sc as plsc

_B_TOTAL = 4096 * 200
_D = 32
_DPAD = 128
_NW = 32
_ROWS_PER_W = 200
_K = 2
_CHUNK = _K * 128
_N_CHUNKS = _ROWS_PER_W // _K


def _transpose_chunk(gbuf, tbuf, voff_lo, voff_hi):
    def r_body(r, _):
        base = (r // 128) * 4096 + (r % 128)
        lo = gbuf[r, pl.ds(0, 16)]
        hi = gbuf[r, pl.ds(16, 16)]
        plsc.store_scatter(tbuf, [voff_lo + base], lo)
        plsc.store_scatter(tbuf, [voff_hi + base], hi)
        return _

    lax.fori_loop(0, _CHUNK, r_body, 0)


def _emb_body(ids_hbm, table_hbm, out_hbm, ids_v, g0, g1, t0, t1, semg, semw):
    wid = lax.axis_index("s") * 2 + lax.axis_index("c")
    base_row = wid * _ROWS_PER_W
    iota16 = jnp.arange(16, dtype=jnp.int32)
    voff_lo = (iota16 // 8) * 1024 + (iota16 % 8) * 128
    voff_hi = voff_lo + 2048

    pltpu.sync_copy(ids_hbm.at[pl.ds(base_row * 128, _ROWS_PER_W * 128)], ids_v)

    def gather(c, buf):
        pltpu.async_copy(
            table_hbm.at[ids_v.at[pl.ds(c * _CHUNK, _CHUNK)]], buf, semg)

    def writeback(c, tbuf):
        for k in range(_K):
            r = base_row + c * _K + k
            st = r // 256
            bt = (r // 8) % 32
            s = st * 8 + r % 8
            for td in range(4):
                pltpu.async_copy(
                    tbuf.at[pl.ds(k * 4096 + td * 1024, 1024)],
                    out_hbm.at[s, td, bt], semw)

    def drain_one_chunk():
        pltpu.make_async_copy(
            table_hbm.at[pl.ds(0, 64)], g0.at[pl.ds(0, 64)], semw).wait()

    def wait_gather(buf):
        pltpu.make_async_copy(
            table_hbm.at[ids_v.at[pl.ds(0, _CHUNK)]], buf, semg).wait()

    def process(c, gbuf, tbuf):
        wait_gather(gbuf)
        _transpose_chunk(gbuf, tbuf, voff_lo, voff_hi)
        writeback(c, tbuf)

    gather(0, g0)
    gather(1, g1)
    process(0, g0, t0)
    gather(2, g0)
    process(1, g1, t1)
    gather(3, g1)

    def steady(m, carry):
        c0 = 2 * m + 2
        drain_one_chunk()
        process(c0, g0, t0)
        gather(c0 + 2, g0)
        drain_one_chunk()
        process(c0 + 1, g1, t1)
        gather(c0 + 3, g1)
        return carry

    lax.fori_loop(0, (_N_CHUNKS - 4) // 2, steady, 0)

    drain_one_chunk()
    process(_N_CHUNKS - 2, g0, t0)
    drain_one_chunk()
    process(_N_CHUNKS - 1, g1, t1)
    drain_one_chunk()
    drain_one_chunk()


@jax.jit
def _embedding_lookup(ids_flat, table_pad):
    mesh = plsc.VectorSubcoreMesh(core_axis_name="c", subcore_axis_name="s")
    f = functools.partial(
        pl.kernel,
        mesh=mesh,
        out_type=jax.ShapeDtypeStruct((200, 4, 32, 1024), jnp.float32),
        scratch_types=[
            pltpu.VMEM((_ROWS_PER_W * 128,), jnp.int32),
            pltpu.VMEM((_CHUNK, _DPAD), jnp.float32),
            pltpu.VMEM((_CHUNK, _DPAD), jnp.float32),
            pltpu.VMEM((_K * 4096,), jnp.float32),
            pltpu.VMEM((_K * 4096,), jnp.float32),
            pltpu.SemaphoreType.DMA,
            pltpu.SemaphoreType.DMA,
        ],
        compiler_params=pltpu.CompilerParams(
            use_tc_tiling_on_sc=False, needs_layout_passes=False),
    )(_emb_body)
    return f(ids_flat, table_pad)


def kernel(input_ids, weight):
    ids1 = (
        input_ids.T.reshape(25, 8, 32, 128)
        .transpose(0, 2, 1, 3)
        .reshape(_B_TOTAL)
        .astype(jnp.int32)
    )
    w_pad = jnp.pad(weight, ((0, 0), (0, _DPAD - _D)))
    out4 = _embedding_lookup(ids1, w_pad)
    out5 = out4.reshape(200, 4, 32, 8, 128)
    return jnp.transpose(out5, (2, 4, 0, 1, 3)).reshape(4096, 200, 32)


# speedup vs baseline: 1.2482x; 1.0808x over previous
import functools

import jax
import jax.numpy as jnp
from jax import lax
from jax.experimental import pallas as pl
from jax.experimental.pallas import tpu as pltpu
from jax.experimental.pallas import tpu_sc as plsc

_B_TOTAL = 4096 * 200
_D = 32
_DPAD = 128
_NW = 32
_ROWS_PER_W = 200
_K = 2
_CHUNK = _K * 128
_N_CHUNKS = _ROWS_PER_W // _K


def _transpose_chunk(gbuf, tbuf, voff_lo, voff_hi):
    @plsc.parallel_loop(0, _CHUNK, step=1, unroll=8)
    def r_body(r):
        base = (r // 128) * 4096 + (r % 128)
        lo = gbuf[r, pl.ds(0, 16)]
        hi = gbuf[r, pl.ds(16, 16)]
        plsc.store_scatter(tbuf, [voff_lo + base], lo)
        plsc.store_scatter(tbuf, [voff_hi + base], hi)


def _emb_body(ids_hbm, table_hbm, out_hbm, ids_v, g0, g1, t0, t1, semg, semw):
    wid = lax.axis_index("s") * 2 + lax.axis_index("c")
    base_row = wid * _ROWS_PER_W
    iota16 = jnp.arange(16, dtype=jnp.int32)
    voff_lo = (iota16 // 8) * 1024 + (iota16 % 8) * 128
    voff_hi = voff_lo + 2048

    pltpu.sync_copy(ids_hbm.at[pl.ds(base_row * 128, _ROWS_PER_W * 128)], ids_v)

    def gather(c, buf):
        pltpu.async_copy(
            table_hbm.at[ids_v.at[pl.ds(c * _CHUNK, _CHUNK)]], buf, semg)

    def writeback(c, tbuf):
        for k in range(_K):
            r = base_row + c * _K + k
            st = r // 256
            bt = (r // 8) % 32
            s = st * 8 + r % 8
            for td in range(4):
                pltpu.async_copy(
                    tbuf.at[pl.ds(k * 4096 + td * 1024, 1024)],
                    out_hbm.at[s, td, bt], semw)

    def drain_one_chunk():
        pltpu.make_async_copy(
            table_hbm.at[pl.ds(0, 64)], g0.at[pl.ds(0, 64)], semw).wait()

    def wait_gather(buf):
        pltpu.make_async_copy(
            table_hbm.at[ids_v.at[pl.ds(0, _CHUNK)]], buf, semg).wait()

    def process(c, gbuf, tbuf):
        wait_gather(gbuf)
        _transpose_chunk(gbuf, tbuf, voff_lo, voff_hi)
        writeback(c, tbuf)

    gather(0, g0)
    gather(1, g1)
    process(0, g0, t0)
    gather(2, g0)
    process(1, g1, t1)
    gather(3, g1)

    def steady(m, carry):
        c0 = 2 * m + 2
        drain_one_chunk()
        process(c0, g0, t0)
        gather(c0 + 2, g0)
        drain_one_chunk()
        process(c0 + 1, g1, t1)
        gather(c0 + 3, g1)
        return carry

    lax.fori_loop(0, (_N_CHUNKS - 4) // 2, steady, 0)

    drain_one_chunk()
    process(_N_CHUNKS - 2, g0, t0)
    drain_one_chunk()
    process(_N_CHUNKS - 1, g1, t1)
    drain_one_chunk()
    drain_one_chunk()


@jax.jit
def _embedding_lookup(ids_flat, table_pad):
    mesh = plsc.VectorSubcoreMesh(core_axis_name="c", subcore_axis_name="s")
    f = functools.partial(
        pl.kernel,
        mesh=mesh,
        out_type=jax.ShapeDtypeStruct((200, 4, 32, 1024), jnp.float32),
        scratch_types=[
            pltpu.VMEM((_ROWS_PER_W * 128,), jnp.int32),
            pltpu.VMEM((_CHUNK, _DPAD), jnp.float32),
            pltpu.VMEM((_CHUNK, _DPAD), jnp.float32),
            pltpu.VMEM((_K * 4096,), jnp.float32),
            pltpu.VMEM((_K * 4096,), jnp.float32),
            pltpu.SemaphoreType.DMA,
            pltpu.SemaphoreType.DMA,
        ],
        compiler_params=pltpu.CompilerParams(
            use_tc_tiling_on_sc=False, needs_layout_passes=False),
    )(_emb_body)
    return f(ids_flat, table_pad)


def kernel(input_ids, weight):
    ids1 = (
        input_ids.T.reshape(25, 8, 32, 128)
        .transpose(0, 2, 1, 3)
        .reshape(_B_TOTAL)
        .astype(jnp.int32)
    )
    w_pad = jnp.pad(weight, ((0, 0), (0, _DPAD - _D)))
    out4 = _embedding_lookup(ids1, w_pad)
    out5 = out4.reshape(200, 4, 32, 8, 128)
    return jnp.transpose(out5, (2, 4, 0, 1, 3)).reshape(4096, 200, 32)


# speedup vs baseline: 1.2492x; 1.0008x over previous
import functools

import jax
import jax.numpy as jnp
from jax import lax
from jax.experimental import pallas as pl
from jax.experimental.pallas import tpu as pltpu
from jax.experimental.pallas import tpu_sc as plsc

_B_TOTAL = 4096 * 200
_D = 32
_DPAD = 128
_NW = 32
_ROWS_PER_W = 200
_K = 2
_CHUNK = _K * 128
_N_CHUNKS = _ROWS_PER_W // _K


def _transpose_chunk(gbuf, tbuf, voff_lo, voff_hi):
    @plsc.parallel_loop(0, _CHUNK, step=1, unroll=16)
    def r_body(r):
        base = (r // 128) * 4096 + (r % 128)
        lo = gbuf[r, pl.ds(0, 16)]
        hi = gbuf[r, pl.ds(16, 16)]
        plsc.store_scatter(tbuf, [voff_lo + base], lo)
        plsc.store_scatter(tbuf, [voff_hi + base], hi)


def _emb_body(ids_hbm, table_hbm, out_hbm, ids_v, g0, g1, t0, t1, semg, semw):
    wid = lax.axis_index("s") * 2 + lax.axis_index("c")
    base_row = wid * _ROWS_PER_W
    iota16 = jnp.arange(16, dtype=jnp.int32)
    voff_lo = (iota16 // 8) * 1024 + (iota16 % 8) * 128
    voff_hi = voff_lo + 2048

    pltpu.sync_copy(ids_hbm.at[pl.ds(base_row * 128, _ROWS_PER_W * 128)], ids_v)

    def gather(c, buf):
        pltpu.async_copy(
            table_hbm.at[ids_v.at[pl.ds(c * _CHUNK, _CHUNK)]], buf, semg)

    def writeback(c, tbuf):
        for k in range(_K):
            r = base_row + c * _K + k
            st = r // 256
            bt = (r // 8) % 32
            s = st * 8 + r % 8
            for td in range(4):
                pltpu.async_copy(
                    tbuf.at[pl.ds(k * 4096 + td * 1024, 1024)],
                    out_hbm.at[s, td, bt], semw)

    def drain_one_chunk():
        pltpu.make_async_copy(
            table_hbm.at[pl.ds(0, 64)], g0.at[pl.ds(0, 64)], semw).wait()

    def wait_gather(buf):
        pltpu.make_async_copy(
            table_hbm.at[ids_v.at[pl.ds(0, _CHUNK)]], buf, semg).wait()

    def process(c, gbuf, tbuf):
        wait_gather(gbuf)
        _transpose_chunk(gbuf, tbuf, voff_lo, voff_hi)
        writeback(c, tbuf)

    gather(0, g0)
    gather(1, g1)
    process(0, g0, t0)
    gather(2, g0)
    process(1, g1, t1)
    gather(3, g1)

    def steady(m, carry):
        c0 = 2 * m + 2
        drain_one_chunk()
        process(c0, g0, t0)
        gather(c0 + 2, g0)
        drain_one_chunk()
        process(c0 + 1, g1, t1)
        gather(c0 + 3, g1)
        return carry

    lax.fori_loop(0, (_N_CHUNKS - 4) // 2, steady, 0)

    drain_one_chunk()
    process(_N_CHUNKS - 2, g0, t0)
    drain_one_chunk()
    process(_N_CHUNKS - 1, g1, t1)
    drain_one_chunk()
    drain_one_chunk()


@jax.jit
def _embedding_lookup(ids_flat, table_pad):
    mesh = plsc.VectorSubcoreMesh(core_axis_name="c", subcore_axis_name="s")
    f = functools.partial(
        pl.kernel,
        mesh=mesh,
        out_type=jax.ShapeDtypeStruct((200, 4, 32, 1024), jnp.float32),
        scratch_types=[
            pltpu.VMEM((_ROWS_PER_W * 128,), jnp.int32),
            pltpu.VMEM((_CHUNK, _DPAD), jnp.float32),
            pltpu.VMEM((_CHUNK, _DPAD), jnp.float32),
            pltpu.VMEM((_K * 4096,), jnp.float32),
            pltpu.VMEM((_K * 4096,), jnp.float32),
            pltpu.SemaphoreType.DMA,
            pltpu.SemaphoreType.DMA,
        ],
        compiler_params=pltpu.CompilerParams(
            use_tc_tiling_on_sc=False, needs_layout_passes=False),
    )(_emb_body)
    return f(ids_flat, table_pad)


def kernel(input_ids, weight):
    ids1 = (
        input_ids.T.reshape(25, 8, 32, 128)
        .transpose(0, 2, 1, 3)
        .reshape(_B_TOTAL)
        .astype(jnp.int32)
    )
    w_pad = jnp.pad(weight, ((0, 0), (0, _DPAD - _D)))
    out4 = _embedding_lookup(ids1, w_pad)
    out5 = out4.reshape(200, 4, 32, 8, 128)
    return jnp.transpose(out5, (2, 4, 0, 1, 3)).reshape(4096, 200, 32)
